# Initial kernel scaffold; baseline (speedup 1.0000x reference)
import jax
import jax.numpy as jnp
from jax.experimental import pallas as pl


def kernel(x1, edge_index1, edge_type1, x2, edge_index2, edge_type2, batch1, batch2, W1_0, Wr1_0, b1_0, W1_1, Wr1_1, b1_1, W2_0, Wr2_0, b2_0, W2_1, Wr2_1, b2_1, Wm1, bm1, Wm2, bm2, Wm3, bm3):
    raise NotImplementedError("write your pallas kernel here")


# speedup vs baseline: 10.8835x; 10.8835x over previous
import functools

import jax
import jax.numpy as jnp
from jax import lax
from jax.experimental import pallas as pl
from jax.experimental.pallas import tpu as pltpu
from jax.experimental.pallas import tpu_sc as plsc

N = 10000
D = 256
R = 8
G = 16
E = 160000
NC = 2
NS = 16
HALF = D // NC
NP = 10112
ROWS_T = NP // NS
ROW_CHUNKS = ((0, 128), (128, 128), (256, 128), (384, 128), (512, 120))
CH = 128
NCHUNK_REAL = E // CH
NCHUNK = 1280
KCH = NCHUNK // NS
SB = 16
EP = NCHUNK * CH
NR = N * R
NR_T = 5008
NR_P = NR_T * NS


@functools.cache
def _mesh():
    return plsc.VectorSubcoreMesh(
        core_axis_name="c", subcore_axis_name="s", num_cores=NC, num_subcores=NS
    )


def _f16(v):
    return jnp.full((16,), v, dtype=jnp.int32)


def _sc_params():
    return pltpu.CompilerParams(needs_layout_passes=False)


def _prep_body(src_h, et_h, dst_h, idx_out, w_out,
               src_v, et_v, idx2_v, w_v, ones_v, slice_v, cnt_sp):
    c = lax.axis_index("c")
    s = lax.axis_index("s")

    @pl.when(c == 0)
    def _():
        base = s * KCH
        pltpu.sync_copy(src_h.at[pl.ds(base, KCH)], src_v)
        pltpu.sync_copy(et_h.at[pl.ds(base, KCH)], et_v)
        pltpu.sync_copy(dst_h.at[pl.ds(base, KCH)], idx2_v)

        def zbody(i, _):
            slice_v[pl.ds(i * 16, 16)] = jnp.zeros((16,), jnp.float32)
            return 0
        lax.fori_loop(0, NR_T // 16, zbody, 0)
        pltpu.sync_copy(slice_v, cnt_sp.at[pl.ds(s * NR_T, NR_T)])

        def obody(i, _):
            ones_v[pl.ds(i * 16, 16)] = jnp.ones((16,), jnp.float32)
            return 0
        lax.fori_loop(0, CH // 16, obody, 0)

        def cbody(k, _):
            def inner(i, _):
                sl = pl.ds(i * 16, 16)
                srcv = src_v[k, sl]
                etv = et_v[k, sl]
                dstv = idx2_v[k, sl]
                src_v[k, sl] = etv * NP + srcv
                idx2_v[k, sl] = dstv * R + etv
                return 0
            lax.fori_loop(0, CH // 16, inner, 0)
            return 0
        lax.fori_loop(0, KCH, cbody, 0)

        plsc.subcore_barrier()

        def hbody(k, _):
            @pl.when(base + k < NCHUNK_REAL)
            def _():
                pltpu.sync_copy(ones_v, cnt_sp.at[idx2_v.at[k]], add=True)
            return 0
        lax.fori_loop(0, KCH, hbody, 0)

        plsc.subcore_barrier()

        pltpu.sync_copy(cnt_sp.at[pl.ds(s * NR_T, NR_T)], slice_v)

        def ibody(i, _):
            sl = pl.ds(i * 16, 16)
            slice_v[sl] = 1.0 / jnp.maximum(slice_v[sl], 1.0)
            return 0
        lax.fori_loop(0, NR_T // 16, ibody, 0)
        pltpu.sync_copy(slice_v, cnt_sp.at[pl.ds(s * NR_T, NR_T)])

        plsc.subcore_barrier()

        def wbody(k, _):
            pltpu.sync_copy(cnt_sp.at[idx2_v.at[k]], w_v.at[k])
            return 0
        lax.fori_loop(0, KCH, wbody, 0)

        def wpad(k, _):
            @pl.when(base + k >= NCHUNK_REAL)
            def _():
                def zrow(i, _):
                    w_v[k, pl.ds(i * 16, 16)] = jnp.zeros((16,), jnp.float32)
                    return 0
                lax.fori_loop(0, CH // 16, zrow, 0)
            return 0
        lax.fori_loop(0, KCH, wpad, 0)

        pltpu.sync_copy(src_v, idx_out.at[pl.ds(base, KCH)])
        pltpu.sync_copy(w_v, w_out.at[pl.ds(base, KCH)])


@functools.cache
def _prep():
    return pl.kernel(
        _prep_body,
        out_type=(
            jax.ShapeDtypeStruct((NCHUNK, CH), jnp.int32),
            jax.ShapeDtypeStruct((NCHUNK, CH), jnp.float32),
        ),
        mesh=_mesh(),
        scratch_types=[
            pltpu.VMEM((KCH, CH), jnp.int32),
            pltpu.VMEM((KCH, CH), jnp.int32),
            pltpu.VMEM((KCH, CH), jnp.int32),
            pltpu.VMEM((KCH, CH), jnp.float32),
            pltpu.VMEM((CH,), jnp.float32),
            pltpu.VMEM((NR_T,), jnp.float32),
            pltpu.VMEM_SHARED((NR_P,), jnp.float32),
        ],
        compiler_params=_sc_params(),
    )


def _make_layer(pool_epilogue):
    if pool_epilogue:
        out_type = jax.ShapeDtypeStruct((NC, NS, G, HALF), jnp.float32)
    else:
        out_type = jax.ShapeDtypeStruct((NC, NP, HALF), jnp.float32)
    scratch = [
        pltpu.VMEM((SB, CH), jnp.int32),
        pltpu.VMEM((SB, CH), jnp.int32),
        pltpu.VMEM((SB, CH), jnp.float32),
        pltpu.VMEM((CH, HALF), jnp.float32),
        pltpu.VMEM_SHARED((NP, HALF), jnp.float32),
    ]
    if pool_epilogue:
        scratch += [
            pltpu.VMEM((G + 1, HALF), jnp.float32),
            pltpu.VMEM((ROWS_T,), jnp.int32),
        ]

    def body(mm_h, idx_h, dst_h, w_h, batch_h, out_h,
             idx_v, dst_v, w_v, rbuf, acc_sp, *rest):
        c = lax.axis_index("c")
        s = lax.axis_index("s")
        rows0 = s * ROWS_T
        root_base = c * (9 * NP) + 8 * NP

        for (r0, sz) in ROW_CHUNKS:
            pltpu.sync_copy(mm_h.at[pl.ds(root_base + rows0 + r0, sz)],
                            rbuf.at[pl.ds(0, sz)])
            pltpu.sync_copy(rbuf.at[pl.ds(0, sz)],
                            acc_sp.at[pl.ds(rows0 + r0, sz)])

        off = jnp.full((16,), c * (9 * NP), dtype=jnp.int32)

        plsc.subcore_barrier()

        def bbody(bi, _):
            eb = s * KCH + bi * SB
            pltpu.sync_copy(idx_h.at[pl.ds(eb, SB)], idx_v)
            pltpu.sync_copy(dst_h.at[pl.ds(eb, SB)], dst_v)
            pltpu.sync_copy(w_h.at[pl.ds(eb, SB)], w_v)

            def offb(k, _):
                def inner(i, _):
                    sl = pl.ds(i * 16, 16)
                    idx_v[k, sl] = idx_v[k, sl] + off
                    return 0
                lax.fori_loop(0, CH // 16, inner, 0)
                return 0
            lax.fori_loop(0, SB, offb, 0)

            def ebody(k, _):
                pltpu.sync_copy(mm_h.at[idx_v.at[k]], rbuf)

                def scale(e, _):
                    wsp = plsc.load_gather(w_v, [_f16(k), _f16(e)])
                    for j in range(HALF // 16):
                        sl = pl.ds(j * 16, 16)
                        rbuf[e, sl] = rbuf[e, sl] * wsp
                    return 0
                lax.fori_loop(0, CH, scale, 0)
                pltpu.sync_copy(rbuf, acc_sp.at[dst_v.at[k]], add=True)
                return 0
            lax.fori_loop(0, SB, ebody, 0)
            return 0
        lax.fori_loop(0, KCH // SB, bbody, 0)

        plsc.subcore_barrier()

        if pool_epilogue:
            pool_v, batch_v = rest
            col16 = lax.iota(jnp.int32, 16)
            for g in range(G + 1):
                for j in range(HALF // 16):
                    pool_v[g, pl.ds(j * 16, 16)] = jnp.zeros((16,), jnp.float32)
            pltpu.sync_copy(batch_h.at[s, 0], batch_v)

            for (r0, sz) in ROW_CHUNKS:
                pltpu.sync_copy(acc_sp.at[pl.ds(rows0 + r0, sz)],
                                rbuf.at[pl.ds(0, sz)])

                def row(rr, _, r0=r0):
                    gv = plsc.load_gather(batch_v, [_f16(r0 + rr)])
                    for j in range(HALF // 16):
                        sl = pl.ds(j * 16, 16)
                        v = jnp.maximum(rbuf[rr, sl], 0.0)
                        plsc.addupdate_scatter(pool_v, [gv, col16 + (j * 16)], v)
                    return 0
                lax.fori_loop(0, sz, row, 0)
            pltpu.sync_copy(pool_v.at[pl.ds(0, G)], out_h.at[c, s])
        else:
            for (r0, sz) in ROW_CHUNKS:
                pltpu.sync_copy(acc_sp.at[pl.ds(rows0 + r0, sz)],
                                rbuf.at[pl.ds(0, sz)])

                def row(rr, _):
                    for j in range(HALF // 16):
                        sl = pl.ds(j * 16, 16)
                        rbuf[rr, sl] = jnp.maximum(rbuf[rr, sl], 0.0)
                    return 0
                lax.fori_loop(0, sz, row, 0)
                pltpu.sync_copy(rbuf.at[pl.ds(0, sz)],
                                out_h.at[c, pl.ds(rows0 + r0, sz)])

    return pl.kernel(body, out_type=out_type, mesh=_mesh(),
                     scratch_types=scratch, compiler_params=_sc_params())


_layer_relu = functools.cache(lambda: _make_layer(False))
_layer_pool = functools.cache(lambda: _make_layer(True))


_BN = 1264


def _mm_body(x_ref, w_ref, b_ref, o_ref):
    acc = jnp.dot(x_ref[...], w_ref[0, 0], preferred_element_type=jnp.float32)
    o_ref[...] = (acc + b_ref[0, 0])[None, None]


def _mm(x, wstk, bstk):
    return pl.pallas_call(
        _mm_body,
        grid=(NP // _BN, NC, 9),
        in_specs=[
            pl.BlockSpec((_BN, D), lambda i, c, j: (i, 0)),
            pl.BlockSpec((1, 1, D, HALF), lambda i, c, j: (c, j, 0, 0)),
            pl.BlockSpec((1, 1, 1, HALF), lambda i, c, j: (c, j, 0, 0)),
        ],
        out_specs=pl.BlockSpec((1, 1, _BN, HALF), lambda i, c, j: (c, j, i, 0)),
        out_shape=jax.ShapeDtypeStruct((NC, 9, NP, HALF), jnp.float32),
    )(x, wstk, bstk)


def _wstk(W, Wr):
    wall = jnp.concatenate([W, Wr[None]], axis=0)
    return wall.reshape(9, D, NC, HALF).transpose(2, 0, 1, 3)


def _bstk(b):
    return jnp.concatenate(
        [jnp.zeros((NC, 8, HALF), jnp.float32), b.reshape(NC, 1, HALF)], axis=1
    ).reshape(NC, 9, 1, HALF)


def _mlp_body(p1, p2, bb1, bb2, w1, v1, w2, v2, w3, v3, o_ref):
    gids = lax.broadcasted_iota(jnp.int32, (G, 1, 1), 0)

    def pooled(p_ref, b_ref):
        ssum = jnp.sum(p_ref[...], axis=1)
        h = jnp.concatenate([ssum[0], ssum[1]], axis=-1)
        cnt = jnp.sum((b_ref[...][None, :, :] == gids).astype(jnp.float32),
                      axis=(1, 2))
        return h / jnp.clip(cnt, 1.0)[:, None]

    h = jnp.concatenate([pooled(p1, bb1), pooled(p2, bb2)], axis=1)
    h = jax.nn.relu(jnp.dot(h, w1[...], preferred_element_type=jnp.float32)
                    + v1[...][None, :])
    h = jax.nn.relu(jnp.dot(h, w2[...], preferred_element_type=jnp.float32)
                    + v2[...][None, :])
    o_ref[...] = (jnp.dot(h, w3[...], preferred_element_type=jnp.float32)
                  + v3[...][None, :])


def _mlp(pool1, pool2, b1, b2, Wm1, bm1, Wm2, bm2, Wm3p, bm3p):
    return pl.pallas_call(
        _mlp_body,
        out_shape=jax.ShapeDtypeStruct((G, HALF), jnp.float32),
    )(pool1, pool2, b1, b2, Wm1, bm1, Wm2, bm2, Wm3p, bm3p)


def _tower(x, ei, et, Wl0, Wr0, b0, Wl1, Wr1, b1, batch):
    epad = EP - E
    src2 = jnp.pad(ei[0], (0, epad)).reshape(NCHUNK, CH)
    dst2 = jnp.pad(ei[1], (0, epad)).reshape(NCHUNK, CH)
    et2 = jnp.pad(et, (0, epad)).reshape(NCHUNK, CH)
    idx2, w2 = _prep()(src2, et2, dst2)
    xp = jnp.pad(x, ((0, NP - N), (0, 0)))
    mm1 = _mm(xp, _wstk(Wl0, Wr0), _bstk(b0)).reshape(NC * 9 * NP, HALF)
    bt2 = jnp.pad(batch, (0, NP - N), constant_values=G).reshape(NS, 1, ROWS_T)
    xn = _layer_relu()(mm1, idx2, dst2, w2, bt2)
    xcat = jnp.concatenate([xn[0], xn[1]], axis=1)
    mm2 = _mm(xcat, _wstk(Wl1, Wr1), _bstk(b1)).reshape(NC * 9 * NP, HALF)
    return _layer_pool()(mm2, idx2, dst2, w2, bt2)


def kernel(x1, edge_index1, edge_type1, x2, edge_index2, edge_type2,
           batch1, batch2,
           W1_0, Wr1_0, b1_0, W1_1, Wr1_1, b1_1,
           W2_0, Wr2_0, b2_0, W2_1, Wr2_1, b2_1,
           Wm1, bm1, Wm2, bm2, Wm3, bm3):
    pool1 = _tower(x1, edge_index1, edge_type1,
                   W1_0, Wr1_0, b1_0, W1_1, Wr1_1, b1_1, batch1)
    pool2 = _tower(x2, edge_index2, edge_type2,
                   W2_0, Wr2_0, b2_0, W2_1, Wr2_1, b2_1, batch2)
    Wm3p = jnp.pad(Wm3, ((0, 0), (0, HALF - 4)))
    bm3p = jnp.pad(bm3, (0, HALF - 4))
    out = _mlp(pool1, pool2,
               batch1.reshape(G, N // G), batch2.reshape(G, N // G),
               Wm1, bm1, Wm2, bm2, Wm3p, bm3p)
    return out[:, :4]


# speedup vs baseline: 13.7378x; 1.2623x over previous
import functools

import jax
import jax.numpy as jnp
from jax import lax
from jax.experimental import pallas as pl
from jax.experimental.pallas import tpu as pltpu
from jax.experimental.pallas import tpu_sc as plsc

N = 10000
D = 256
R = 8
G = 16
E = 160000
NC = 2
NS = 16
HALF = D // NC
NP = 10112
ROWS_T = NP // NS
ROW_CHUNKS = ((0, 128), (128, 128), (256, 128), (384, 128), (512, 120))
CH = 128
NCHUNK_REAL = E // CH
NCHUNK = 1280
KCH = NCHUNK // NS
SB = 16
EP = NCHUNK * CH
NR = N * R
NR_T = 5008
NR_P = NR_T * NS


@functools.cache
def _mesh():
    return plsc.VectorSubcoreMesh(
        core_axis_name="c", subcore_axis_name="s", num_cores=NC, num_subcores=NS
    )


def _f16(v):
    return jnp.full((16,), v, dtype=jnp.int32)


def _sc_params():
    return pltpu.CompilerParams(needs_layout_passes=False)


def _prep_body(src_h, et_h, dst_h, idx_out, w_out,
               src_v, et_v, idx2_v, w_v, ones_v, slice_v, cnt_sp):
    c = lax.axis_index("c")
    s = lax.axis_index("s")

    @pl.when(c == 0)
    def _():
        base = s * KCH
        pltpu.sync_copy(src_h.at[pl.ds(base, KCH)], src_v)
        pltpu.sync_copy(et_h.at[pl.ds(base, KCH)], et_v)
        pltpu.sync_copy(dst_h.at[pl.ds(base, KCH)], idx2_v)

        def zbody(i, _):
            slice_v[pl.ds(i * 16, 16)] = jnp.zeros((16,), jnp.float32)
            return 0
        lax.fori_loop(0, NR_T // 16, zbody, 0)
        pltpu.sync_copy(slice_v, cnt_sp.at[pl.ds(s * NR_T, NR_T)])

        def obody(i, _):
            ones_v[pl.ds(i * 16, 16)] = jnp.ones((16,), jnp.float32)
            return 0
        lax.fori_loop(0, CH // 16, obody, 0)

        def cbody(k, _):
            def inner(i, _):
                sl = pl.ds(i * 16, 16)
                srcv = src_v[k, sl]
                etv = et_v[k, sl]
                dstv = idx2_v[k, sl]
                src_v[k, sl] = etv * NP + srcv
                idx2_v[k, sl] = dstv * R + etv
                return 0
            lax.fori_loop(0, CH // 16, inner, 0)
            return 0
        lax.fori_loop(0, KCH, cbody, 0)

        plsc.subcore_barrier()

        def hbody(k, _):
            @pl.when(base + k < NCHUNK_REAL)
            def _():
                pltpu.sync_copy(ones_v, cnt_sp.at[idx2_v.at[k]], add=True)
            return 0
        lax.fori_loop(0, KCH, hbody, 0)

        plsc.subcore_barrier()

        pltpu.sync_copy(cnt_sp.at[pl.ds(s * NR_T, NR_T)], slice_v)

        def ibody(i, _):
            sl = pl.ds(i * 16, 16)
            slice_v[sl] = 1.0 / jnp.maximum(slice_v[sl], 1.0)
            return 0
        lax.fori_loop(0, NR_T // 16, ibody, 0)
        pltpu.sync_copy(slice_v, cnt_sp.at[pl.ds(s * NR_T, NR_T)])

        plsc.subcore_barrier()

        def wbody(k, _):
            pltpu.sync_copy(cnt_sp.at[idx2_v.at[k]], w_v.at[k])
            return 0
        lax.fori_loop(0, KCH, wbody, 0)

        def wpad(k, _):
            @pl.when(base + k >= NCHUNK_REAL)
            def _():
                def zrow(i, _):
                    w_v[k, pl.ds(i * 16, 16)] = jnp.zeros((16,), jnp.float32)
                    return 0
                lax.fori_loop(0, CH // 16, zrow, 0)
            return 0
        lax.fori_loop(0, KCH, wpad, 0)

        pltpu.sync_copy(src_v, idx_out.at[pl.ds(base, KCH)])
        pltpu.sync_copy(w_v, w_out.at[pl.ds(base, KCH)])


@functools.cache
def _prep():
    return pl.kernel(
        _prep_body,
        out_type=(
            jax.ShapeDtypeStruct((NCHUNK, CH), jnp.int32),
            jax.ShapeDtypeStruct((NCHUNK, CH), jnp.float32),
        ),
        mesh=_mesh(),
        scratch_types=[
            pltpu.VMEM((KCH, CH), jnp.int32),
            pltpu.VMEM((KCH, CH), jnp.int32),
            pltpu.VMEM((KCH, CH), jnp.int32),
            pltpu.VMEM((KCH, CH), jnp.float32),
            pltpu.VMEM((CH,), jnp.float32),
            pltpu.VMEM((NR_T,), jnp.float32),
            pltpu.VMEM_SHARED((NR_P,), jnp.float32),
        ],
        compiler_params=_sc_params(),
    )


def _make_layer(pool_epilogue):
    if pool_epilogue:
        out_type = jax.ShapeDtypeStruct((NC, NS, G, HALF), jnp.float32)
    else:
        out_type = jax.ShapeDtypeStruct((NC, NP, HALF), jnp.float32)
    scratch = [
        pltpu.VMEM((SB, CH), jnp.int32),
        pltpu.VMEM((SB, CH), jnp.int32),
        pltpu.VMEM((SB, CH), jnp.float32),
        pltpu.VMEM((CH, HALF), jnp.float32),
        pltpu.VMEM_SHARED((NP, HALF), jnp.float32),
        pltpu.VMEM((CH, HALF), jnp.float32),
        pltpu.SemaphoreType.DMA,
        pltpu.SemaphoreType.DMA,
        pltpu.SemaphoreType.DMA,
        pltpu.SemaphoreType.DMA,
    ]
    if pool_epilogue:
        scratch += [
            pltpu.VMEM((G + 1, HALF), jnp.float32),
            pltpu.VMEM((ROWS_T,), jnp.int32),
        ]

    def body(mm_h, idx_h, dst_h, w_h, batch_h, out_h,
             idx_v, dst_v, w_v, rbuf, acc_sp,
             rbuf2, sem_ga, sem_gb, sem_sa, sem_sb, *rest):
        c = lax.axis_index("c")
        s = lax.axis_index("s")
        rows0 = s * ROWS_T
        root_base = c * (9 * NP) + 8 * NP

        for (r0, sz) in ROW_CHUNKS:
            pltpu.sync_copy(mm_h.at[pl.ds(root_base + rows0 + r0, sz)],
                            rbuf.at[pl.ds(0, sz)])
            pltpu.sync_copy(rbuf.at[pl.ds(0, sz)],
                            acc_sp.at[pl.ds(rows0 + r0, sz)])

        off = jnp.full((16,), c * (9 * NP), dtype=jnp.int32)

        plsc.subcore_barrier()

        bufs = (rbuf, rbuf2)
        gsem = (sem_ga, sem_gb)
        ssem = (sem_sa, sem_sb)

        def bbody(bi, _):
            eb = s * KCH + bi * SB
            pltpu.sync_copy(idx_h.at[pl.ds(eb, SB)], idx_v)
            pltpu.sync_copy(dst_h.at[pl.ds(eb, SB)], dst_v)
            pltpu.sync_copy(w_h.at[pl.ds(eb, SB)], w_v)

            def offb(k, _):
                def inner(i, _):
                    sl = pl.ds(i * 16, 16)
                    idx_v[k, sl] = idx_v[k, sl] + off
                    return 0
                lax.fori_loop(0, CH // 16, inner, 0)
                return 0
            lax.fori_loop(0, SB, offb, 0)

            gh = [None, None]
            sh = [None, None]
            gh[0] = pltpu.async_copy(mm_h.at[idx_v.at[0]], bufs[0], gsem[0])
            for k in range(SB):
                b = k & 1
                gh[b].wait()
                if k + 1 < SB:
                    o = 1 - b
                    if sh[o] is not None:
                        sh[o].wait()
                    gh[o] = pltpu.async_copy(mm_h.at[idx_v.at[k + 1]],
                                             bufs[o], gsem[o])

                def scale(e, _, k=k, b=b):
                    wsp = plsc.load_gather(w_v, [_f16(k), _f16(e)])
                    for j in range(HALF // 16):
                        sl = pl.ds(j * 16, 16)
                        bufs[b][e, sl] = bufs[b][e, sl] * wsp
                    return 0
                lax.fori_loop(0, CH, scale, 0)
                sh[b] = pltpu.async_copy(bufs[b], acc_sp.at[dst_v.at[k]],
                                         ssem[b], add=True)
            sh[0].wait()
            sh[1].wait()
            return 0
        lax.fori_loop(0, KCH // SB, bbody, 0)

        plsc.subcore_barrier()

        if pool_epilogue:
            pool_v, batch_v = rest
            col16 = lax.iota(jnp.int32, 16)
            for g in range(G + 1):
                for j in range(HALF // 16):
                    pool_v[g, pl.ds(j * 16, 16)] = jnp.zeros((16,), jnp.float32)
            pltpu.sync_copy(batch_h.at[s, 0], batch_v)

            for (r0, sz) in ROW_CHUNKS:
                pltpu.sync_copy(acc_sp.at[pl.ds(rows0 + r0, sz)],
                                rbuf.at[pl.ds(0, sz)])

                def row(rr, _, r0=r0):
                    gv = plsc.load_gather(batch_v, [_f16(r0 + rr)])
                    for j in range(HALF // 16):
                        sl = pl.ds(j * 16, 16)
                        v = jnp.maximum(rbuf[rr, sl], 0.0)
                        plsc.addupdate_scatter(pool_v, [gv, col16 + (j * 16)], v)
                    return 0
                lax.fori_loop(0, sz, row, 0)
            pltpu.sync_copy(pool_v.at[pl.ds(0, G)], out_h.at[c, s])
        else:
            for (r0, sz) in ROW_CHUNKS:
                pltpu.sync_copy(acc_sp.at[pl.ds(rows0 + r0, sz)],
                                rbuf.at[pl.ds(0, sz)])

                def row(rr, _):
                    for j in range(HALF // 16):
                        sl = pl.ds(j * 16, 16)
                        rbuf[rr, sl] = jnp.maximum(rbuf[rr, sl], 0.0)
                    return 0
                lax.fori_loop(0, sz, row, 0)
                pltpu.sync_copy(rbuf.at[pl.ds(0, sz)],
                                out_h.at[c, pl.ds(rows0 + r0, sz)])

    return pl.kernel(body, out_type=out_type, mesh=_mesh(),
                     scratch_types=scratch, compiler_params=_sc_params())


_layer_relu = functools.cache(lambda: _make_layer(False))
_layer_pool = functools.cache(lambda: _make_layer(True))


_BN = 1264


def _mm_body(x_ref, w_ref, b_ref, o_ref):
    acc = jnp.dot(x_ref[...], w_ref[0, 0], preferred_element_type=jnp.float32)
    o_ref[...] = (acc + b_ref[0, 0])[None, None]


def _mm(x, wstk, bstk):
    return pl.pallas_call(
        _mm_body,
        grid=(NP // _BN, NC, 9),
        in_specs=[
            pl.BlockSpec((_BN, D), lambda i, c, j: (i, 0)),
            pl.BlockSpec((1, 1, D, HALF), lambda i, c, j: (c, j, 0, 0)),
            pl.BlockSpec((1, 1, 1, HALF), lambda i, c, j: (c, j, 0, 0)),
        ],
        out_specs=pl.BlockSpec((1, 1, _BN, HALF), lambda i, c, j: (c, j, i, 0)),
        out_shape=jax.ShapeDtypeStruct((NC, 9, NP, HALF), jnp.float32),
    )(x, wstk, bstk)


def _wstk(W, Wr):
    wall = jnp.concatenate([W, Wr[None]], axis=0)
    return wall.reshape(9, D, NC, HALF).transpose(2, 0, 1, 3)


def _bstk(b):
    return jnp.concatenate(
        [jnp.zeros((NC, 8, HALF), jnp.float32), b.reshape(NC, 1, HALF)], axis=1
    ).reshape(NC, 9, 1, HALF)


def _mlp_body(p1, p2, bb1, bb2, w1, v1, w2, v2, w3, v3, o_ref):
    gids = lax.broadcasted_iota(jnp.int32, (G, 1, 1), 0)

    def pooled(p_ref, b_ref):
        ssum = jnp.sum(p_ref[...], axis=1)
        h = jnp.concatenate([ssum[0], ssum[1]], axis=-1)
        cnt = jnp.sum((b_ref[...][None, :, :] == gids).astype(jnp.float32),
                      axis=(1, 2))
        return h / jnp.clip(cnt, 1.0)[:, None]

    h = jnp.concatenate([pooled(p1, bb1), pooled(p2, bb2)], axis=1)
    h = jax.nn.relu(jnp.dot(h, w1[...], preferred_element_type=jnp.float32)
                    + v1[...][None, :])
    h = jax.nn.relu(jnp.dot(h, w2[...], preferred_element_type=jnp.float32)
                    + v2[...][None, :])
    o_ref[...] = (jnp.dot(h, w3[...], preferred_element_type=jnp.float32)
                  + v3[...][None, :])


def _mlp(pool1, pool2, b1, b2, Wm1, bm1, Wm2, bm2, Wm3p, bm3p):
    return pl.pallas_call(
        _mlp_body,
        out_shape=jax.ShapeDtypeStruct((G, HALF), jnp.float32),
    )(pool1, pool2, b1, b2, Wm1, bm1, Wm2, bm2, Wm3p, bm3p)


def _tower(x, ei, et, Wl0, Wr0, b0, Wl1, Wr1, b1, batch):
    epad = EP - E
    src2 = jnp.pad(ei[0], (0, epad)).reshape(NCHUNK, CH)
    dst2 = jnp.pad(ei[1], (0, epad)).reshape(NCHUNK, CH)
    et2 = jnp.pad(et, (0, epad)).reshape(NCHUNK, CH)
    idx2, w2 = _prep()(src2, et2, dst2)
    xp = jnp.pad(x, ((0, NP - N), (0, 0)))
    mm1 = _mm(xp, _wstk(Wl0, Wr0), _bstk(b0)).reshape(NC * 9 * NP, HALF)
    bt2 = jnp.pad(batch, (0, NP - N), constant_values=G).reshape(NS, 1, ROWS_T)
    xn = _layer_relu()(mm1, idx2, dst2, w2, bt2)
    xcat = jnp.concatenate([xn[0], xn[1]], axis=1)
    mm2 = _mm(xcat, _wstk(Wl1, Wr1), _bstk(b1)).reshape(NC * 9 * NP, HALF)
    return _layer_pool()(mm2, idx2, dst2, w2, bt2)


def kernel(x1, edge_index1, edge_type1, x2, edge_index2, edge_type2,
           batch1, batch2,
           W1_0, Wr1_0, b1_0, W1_1, Wr1_1, b1_1,
           W2_0, Wr2_0, b2_0, W2_1, Wr2_1, b2_1,
           Wm1, bm1, Wm2, bm2, Wm3, bm3):
    pool1 = _tower(x1, edge_index1, edge_type1,
                   W1_0, Wr1_0, b1_0, W1_1, Wr1_1, b1_1, batch1)
    pool2 = _tower(x2, edge_index2, edge_type2,
                   W2_0, Wr2_0, b2_0, W2_1, Wr2_1, b2_1, batch2)
    Wm3p = jnp.pad(Wm3, ((0, 0), (0, HALF - 4)))
    bm3p = jnp.pad(bm3, (0, HALF - 4))
    out = _mlp(pool1, pool2,
               batch1.reshape(G, N // G), batch2.reshape(G, N // G),
               Wm1, bm1, Wm2, bm2, Wm3p, bm3p)
    return out[:, :4]
